# baseline (speedup 1.0000x reference)
import functools

import jax
import jax.numpy as jnp
from jax import lax
from jax.experimental import pallas as pl
from jax.experimental.pallas import tpu as pltpu
from jax.experimental.pallas import tpu_sc as plsc

EPS = 1e-6
ROPE_THETA = 1000000.0
TOP_K = 8
CAPACITY = 512
BQ = 256
BN_QKV = 512
BM_O = 256
CCHUNK = 256


def _qkv_body(x_ref, ln_ref, w_ref, o_ref):
    x = x_ref[...]
    var = jnp.mean(jnp.square(x), axis=1, keepdims=True)
    xn = (x * lax.rsqrt(var + EPS) * ln_ref[...]).astype(jnp.bfloat16)
    o_ref[...] = lax.dot_general(xn, w_ref[...], (((1,), (1,)), ((), ())),
                                 preferred_element_type=jnp.float32)


def _attn_body(q_ref, k_ref, v_ref, o_ref, *, scale, bq, t):
    i = pl.program_id(1)
    q = q_ref[...]
    k = k_ref[...]
    s = lax.dot_general(q, k, (((1,), (1,)), ((), ())),
                        preferred_element_type=jnp.float32) * scale
    row = lax.broadcasted_iota(jnp.int32, (bq, t), 0) + i * bq
    col = lax.broadcasted_iota(jnp.int32, (bq, t), 1)
    s = jnp.where(row >= col, s, -1e30)
    m = jnp.max(s, axis=1, keepdims=True)
    p = jnp.exp(s - m)
    l = jnp.sum(p, axis=1, keepdims=True)
    o = lax.dot_general(p.astype(jnp.bfloat16), v_ref[...], (((1,), (0,)), ((), ())),
                        preferred_element_type=jnp.float32)
    o_ref[...] = (o / l).astype(jnp.bfloat16)


def _oproj_body(x_ref, w_ref, r_ref, o_ref):
    o_ref[...] = lax.dot_general(x_ref[...], w_ref[...], (((1,), (1,)), ((), ())),
                                 preferred_element_type=jnp.float32) + r_ref[...]


def _normgate_body(x_ref, ln_ref, gw_ref, h_ref, lg_ref):
    x = x_ref[...]
    var = jnp.mean(jnp.square(x), axis=1, keepdims=True)
    xn = x * lax.rsqrt(var + EPS) * ln_ref[...]
    h_ref[...] = xn
    lg_ref[...] = lax.dot_general(xn, gw_ref[...], (((1,), (1,)), ((), ())),
                                  preferred_element_type=jnp.float32)


def _moe_body(tok_ref, val_ref, xe_ref, wg_ref, wu_ref, wd_ref, o_ref, yd_s,
              *, t):
    e = pl.program_id(0)
    jf = pl.program_id(1)

    @pl.when((e == 0) & (jf == 0))
    def _():
        o_ref[...] = jnp.zeros(o_ref.shape, jnp.float32)

    tok = tok_ref[0, 0, 0, :]
    cap = tok.shape[0]
    xe = xe_ref[...]
    g = lax.dot_general(xe, wg_ref[0].astype(jnp.bfloat16), (((1,), (0,)), ((), ())),
                        preferred_element_type=jnp.float32)
    u = lax.dot_general(xe, wu_ref[0].astype(jnp.bfloat16), (((1,), (0,)), ((), ())),
                        preferred_element_type=jnp.float32)
    act = ((g / (1.0 + jnp.exp(-g))) * u).astype(jnp.bfloat16)
    ydp = lax.dot_general(act, wd_ref[0].astype(jnp.bfloat16), (((1,), (0,)), ((), ())),
                          preferred_element_type=jnp.float32)

    @pl.when(jf == 0)
    def _():
        yd_s[...] = ydp

    @pl.when(jf == 1)
    def _():
        yd = yd_s[...] + ydp
        vals = val_ref[0, 0, 0, :][:, None]
        scaled = (yd * vals).astype(jnp.bfloat16)
        onehot = (tok[:, None] == lax.broadcasted_iota(jnp.int32, (cap, t), 1)
                  ).astype(jnp.bfloat16)
        o_ref[...] += lax.dot_general(onehot, scaled, (((0,), (0,)), ((), ())),
                                      preferred_element_type=jnp.float32)


def _sc_gather(table, idx, n_rows, row_words):
    info = plsc.get_sparse_core_info()
    nw = info.num_cores * info.num_subcores
    b_pw = n_rows // nw
    ch = 64
    nch = b_pw // ch
    mesh = plsc.VectorSubcoreMesh(core_axis_name="c", subcore_axis_name="s")

    @functools.partial(
        pl.kernel, mesh=mesh,
        out_type=jax.ShapeDtypeStruct((n_rows, row_words), jnp.int32),
        scratch_types=[
            pltpu.VMEM((b_pw,), jnp.int32),
            pltpu.VMEM((ch, row_words), jnp.int32),
            pltpu.SemaphoreType.DMA,
        ],
    )
    def k(table_hbm, idx_hbm, out_hbm, idx_v, rows_v, sem):
        wid = lax.axis_index("s") * info.num_cores + lax.axis_index("c")
        base = wid * b_pw
        pltpu.sync_copy(idx_hbm.at[pl.ds(base, b_pw)], idx_v)

        def body(c, carry):
            pltpu.async_copy(table_hbm.at[idx_v.at[pl.ds(c * ch, ch)]], rows_v,
                             sem).wait()
            pltpu.sync_copy(rows_v, out_hbm.at[pl.ds(base + c * ch, ch)])
            return carry

        lax.fori_loop(0, nch, body, 0)

    return k(table, idx)


def _rmsnorm(x, w):
    var = jnp.mean(jnp.square(x), axis=-1, keepdims=True)
    return x * lax.rsqrt(var + EPS) * w


def _rope(pos, x, head_dim):
    inv_freq = 1.0 / (ROPE_THETA ** (jnp.arange(0, head_dim, 2, dtype=jnp.float32) / head_dim))
    freqs = pos.astype(jnp.float32)[:, None] * inv_freq[None, :]
    cos = jnp.cos(freqs)[:, None, :]
    sin = jnp.sin(freqs)[:, None, :]
    x1, x2 = jnp.split(x, 2, axis=-1)
    return jnp.concatenate([x1 * cos - x2 * sin, x2 * cos + x1 * sin], axis=-1)


def _route(logits, num_experts):
    t = logits.shape[0]
    probs = jax.nn.softmax(logits, axis=-1)
    topk_w, topk_i = lax.top_k(probs, TOP_K)
    topk_w = topk_w / jnp.sum(topk_w, axis=-1, keepdims=True)
    wf = topk_w.reshape(-1)
    ef = topk_i.reshape(-1).astype(jnp.int32)
    slot = jnp.arange(t * TOP_K, dtype=jnp.int32)
    e_s, negw_s, slot_s = lax.sort((ef, -wf, slot), num_keys=2)
    pos = jnp.arange(t * TOP_K, dtype=jnp.int32)
    is_start = jnp.concatenate([jnp.ones((1,), bool), e_s[1:] != e_s[:-1]])
    start_pos = lax.cummax(jnp.where(is_start, pos, 0))
    rank = pos - start_pos
    keep = rank < CAPACITY
    dest = jnp.where(keep, e_s * CAPACITY + rank, jnp.int32(2 ** 30))
    tok_all = jnp.zeros((num_experts * CAPACITY,), jnp.int32).at[dest].set(slot_s // TOP_K)
    val_all = jnp.zeros((num_experts * CAPACITY,), jnp.float32).at[dest].set(-negw_s)
    return tok_all, val_all


def kernel(hidden_states, ln1_w, qkv_w, q_norm_w, k_norm_w, o_proj_w, ln2_w,
           gate_w, w_gate_up, w_down, positions):
    t, d = hidden_states.shape
    n_qkv = qkv_w.shape[0]
    q_size = o_proj_w.shape[1]
    kv_size = (n_qkv - q_size) // 2
    head_dim = q_norm_w.shape[0]
    num_heads = q_size // head_dim
    num_kv = kv_size // head_dim
    rep = num_heads // num_kv
    num_experts = gate_w.shape[0]
    ff = w_gate_up.shape[2] // 2
    f32 = jnp.float32

    qkv = pl.pallas_call(
        _qkv_body,
        grid=(n_qkv // BN_QKV,),
        in_specs=[
            pl.BlockSpec((t, d), lambda j: (0, 0)),
            pl.BlockSpec((1, d), lambda j: (0, 0)),
            pl.BlockSpec((BN_QKV, d), lambda j: (j, 0)),
        ],
        out_specs=pl.BlockSpec((t, BN_QKV), lambda j: (0, j)),
        out_shape=jax.ShapeDtypeStruct((t, n_qkv), f32),
    )(hidden_states, ln1_w.reshape(1, d), qkv_w.astype(jnp.bfloat16))

    q = qkv[:, :q_size].reshape(t, num_heads, head_dim)
    k = qkv[:, q_size:q_size + kv_size].reshape(t, num_kv, head_dim)
    v2 = qkv[:, q_size + kv_size:].astype(jnp.bfloat16)
    q = _rope(positions, _rmsnorm(q, q_norm_w), head_dim).reshape(t, q_size).astype(jnp.bfloat16)
    k = _rope(positions, _rmsnorm(k, k_norm_w), head_dim).reshape(t, kv_size).astype(jnp.bfloat16)

    attn = pl.pallas_call(
        functools.partial(_attn_body, scale=head_dim ** -0.5, bq=BQ, t=t),
        grid=(num_heads, t // BQ),
        in_specs=[
            pl.BlockSpec((BQ, head_dim), lambda h, i: (i, h)),
            pl.BlockSpec((t, head_dim), lambda h, i: (0, h // rep)),
            pl.BlockSpec((t, head_dim), lambda h, i: (0, h // rep)),
        ],
        out_specs=pl.BlockSpec((BQ, head_dim), lambda h, i: (i, h)),
        out_shape=jax.ShapeDtypeStruct((t, q_size), jnp.bfloat16),
    )(q, k, v2)

    h_attn = pl.pallas_call(
        _oproj_body,
        grid=(t // BM_O,),
        in_specs=[
            pl.BlockSpec((BM_O, q_size), lambda i: (i, 0)),
            pl.BlockSpec((d, q_size), lambda i: (0, 0)),
            pl.BlockSpec((BM_O, d), lambda i: (i, 0)),
        ],
        out_specs=pl.BlockSpec((BM_O, d), lambda i: (i, 0)),
        out_shape=jax.ShapeDtypeStruct((t, d), f32),
    )(attn, o_proj_w.astype(jnp.bfloat16), hidden_states)

    h2, logits = pl.pallas_call(
        _normgate_body,
        grid=(1,),
        in_specs=[
            pl.BlockSpec((t, d), lambda j: (0, 0)),
            pl.BlockSpec((1, d), lambda j: (0, 0)),
            pl.BlockSpec((num_experts, d), lambda j: (0, 0)),
        ],
        out_specs=[
            pl.BlockSpec((t, d), lambda j: (0, 0)),
            pl.BlockSpec((t, num_experts), lambda j: (0, 0)),
        ],
        out_shape=[
            jax.ShapeDtypeStruct((t, d), f32),
            jax.ShapeDtypeStruct((t, num_experts), f32),
        ],
    )(h_attn, ln2_w.reshape(1, d), gate_w)

    tok_all, val_all = _route(logits, num_experts)
    tok4 = tok_all.reshape(num_experts, 1, 1, CAPACITY)
    val4 = val_all.reshape(num_experts, 1, 1, CAPACITY)

    ffc = ff // 2

    nr = num_experts * CAPACITY
    h2i = lax.bitcast_convert_type(
        h2.astype(jnp.bfloat16).reshape(t, d // 2, 2), jnp.int32)
    xei = _sc_gather(h2i, tok_all, nr, d // 2)
    xe_all = lax.bitcast_convert_type(xei, jnp.bfloat16).reshape(nr, d)

    moe = pl.pallas_call(
        functools.partial(_moe_body, t=t),
        grid=(num_experts, 2),
        in_specs=[
            pl.BlockSpec((1, 1, 1, CAPACITY), lambda e, j: (e, 0, 0, 0)),
            pl.BlockSpec((1, 1, 1, CAPACITY), lambda e, j: (e, 0, 0, 0)),
            pl.BlockSpec((CAPACITY, d), lambda e, j: (e, 0)),
            pl.BlockSpec((1, d, ffc), lambda e, j: (e, 0, j)),
            pl.BlockSpec((1, d, ffc), lambda e, j: (e, 0, 2 + j)),
            pl.BlockSpec((1, ffc, d), lambda e, j: (e, j, 0)),
        ],
        out_specs=pl.BlockSpec((t, d), lambda e, j: (0, 0)),
        out_shape=jax.ShapeDtypeStruct((t, d), f32),
        scratch_shapes=[
            pltpu.VMEM((CAPACITY, d), jnp.float32),
        ],
    )(tok4, val4, xe_all, w_gate_up, w_gate_up, w_down)

    return moe + h_attn


# speedup vs baseline: 1.7980x; 1.7980x over previous
import functools

import jax
import jax.numpy as jnp
from jax import lax
from jax.experimental import pallas as pl
from jax.experimental.pallas import tpu as pltpu

EPS = 1e-6
ROPE_THETA = 1000000.0
TOP_K = 8
CAPACITY = 512
BQ = 256
BN_QKV = 512
BM_O = 256
CCHUNK = 256


def _qkv_body(x_ref, ln_ref, w_ref, o_ref):
    x = x_ref[...]
    var = jnp.mean(jnp.square(x), axis=1, keepdims=True)
    xn = (x * lax.rsqrt(var + EPS) * ln_ref[...]).astype(jnp.bfloat16)
    o_ref[...] = lax.dot_general(xn, w_ref[...], (((1,), (1,)), ((), ())),
                                 preferred_element_type=jnp.float32)


def _attn_body(q_ref, qn_ref, cos_ref, sin_ref, k_ref, v_ref, o_ref, *,
               scale, bq, t, hd):
    i = pl.program_id(1)
    qr = q_ref[...]
    var = jnp.mean(jnp.square(qr), axis=1, keepdims=True)
    qn = qr * lax.rsqrt(var + EPS) * qn_ref[...]
    x1 = qn[:, :hd // 2]
    x2 = qn[:, hd // 2:]
    cos = cos_ref[...]
    sin = sin_ref[...]
    q = jnp.concatenate([x1 * cos - x2 * sin, x2 * cos + x1 * sin],
                        axis=1).astype(jnp.bfloat16)
    k = k_ref[...]
    s = lax.dot_general(q, k, (((1,), (1,)), ((), ())),
                        preferred_element_type=jnp.float32) * scale
    row = lax.broadcasted_iota(jnp.int32, (bq, t), 0) + i * bq
    col = lax.broadcasted_iota(jnp.int32, (bq, t), 1)
    s = jnp.where(row >= col, s, -1e30)
    m = jnp.max(s, axis=1, keepdims=True)
    p = jnp.exp(s - m)
    l = jnp.sum(p, axis=1, keepdims=True)
    o = lax.dot_general(p.astype(jnp.bfloat16), v_ref[...], (((1,), (0,)), ((), ())),
                        preferred_element_type=jnp.float32)
    o_ref[...] = (o / l).astype(jnp.bfloat16)


def _oproj_body(x_ref, w_ref, r_ref, o_ref):
    o_ref[...] = lax.dot_general(x_ref[...], w_ref[...], (((1,), (1,)), ((), ())),
                                 preferred_element_type=jnp.float32) + r_ref[...]


def _normgate_body(x_ref, ln_ref, gw_ref, h_ref, lg_ref):
    x = x_ref[...]
    var = jnp.mean(jnp.square(x), axis=1, keepdims=True)
    xn = x * lax.rsqrt(var + EPS) * ln_ref[...]
    h_ref[...] = xn
    lg_ref[...] = lax.dot_general(xn, gw_ref[...], (((1,), (1,)), ((), ())),
                                  preferred_element_type=jnp.float32)


def _moe_body(tok_ref, val_ref, h_ref, wg_ref, wu_ref, wd_ref, o_ref, xe_s, yd_s,
              *, t):
    e = pl.program_id(0)
    jf = pl.program_id(1)

    @pl.when((e == 0) & (jf == 0))
    def _():
        o_ref[...] = jnp.zeros(o_ref.shape, jnp.float32)

    tok = tok_ref[0, 0, 0, :]
    cap = tok.shape[0]

    @pl.when(jf == 0)
    def _():
        onehot = (tok[:, None] == lax.broadcasted_iota(jnp.int32, (cap, t), 1)
                  ).astype(jnp.bfloat16)
        xe_s[...] = lax.dot_general(onehot, h_ref[...], (((1,), (0,)), ((), ())),
                                    preferred_element_type=jnp.float32
                                    ).astype(jnp.bfloat16)

    xe = xe_s[...]
    g = lax.dot_general(xe, wg_ref[0].astype(jnp.bfloat16), (((1,), (0,)), ((), ())),
                        preferred_element_type=jnp.float32)
    u = lax.dot_general(xe, wu_ref[0].astype(jnp.bfloat16), (((1,), (0,)), ((), ())),
                        preferred_element_type=jnp.float32)
    act = ((g / (1.0 + jnp.exp(-g))) * u).astype(jnp.bfloat16)
    ydp = lax.dot_general(act, wd_ref[0].astype(jnp.bfloat16), (((1,), (0,)), ((), ())),
                          preferred_element_type=jnp.float32)

    @pl.when(jf == 0)
    def _():
        yd_s[...] = ydp

    @pl.when(jf == 1)
    def _():
        yd = yd_s[...] + ydp
        vals = val_ref[0, 0, 0, :][:, None]
        scaled = (yd * vals).astype(jnp.bfloat16)
        onehot = (tok[:, None] == lax.broadcasted_iota(jnp.int32, (cap, t), 1)
                  ).astype(jnp.bfloat16)
        o_ref[...] += lax.dot_general(onehot, scaled, (((0,), (0,)), ((), ())),
                                      preferred_element_type=jnp.float32)


def _rmsnorm(x, w):
    var = jnp.mean(jnp.square(x), axis=-1, keepdims=True)
    return x * lax.rsqrt(var + EPS) * w


def _rope(pos, x, head_dim):
    inv_freq = 1.0 / (ROPE_THETA ** (jnp.arange(0, head_dim, 2, dtype=jnp.float32) / head_dim))
    freqs = pos.astype(jnp.float32)[:, None] * inv_freq[None, :]
    cos = jnp.cos(freqs)[:, None, :]
    sin = jnp.sin(freqs)[:, None, :]
    x1, x2 = jnp.split(x, 2, axis=-1)
    return jnp.concatenate([x1 * cos - x2 * sin, x2 * cos + x1 * sin], axis=-1)


def _route(logits, num_experts):
    t = logits.shape[0]
    probs = jax.nn.softmax(logits, axis=-1)
    topk_w, topk_i = lax.top_k(probs, TOP_K)
    topk_w = topk_w / jnp.sum(topk_w, axis=-1, keepdims=True)
    wf = topk_w.reshape(-1)
    ef = topk_i.reshape(-1).astype(jnp.int32)
    slot = jnp.arange(t * TOP_K, dtype=jnp.int32)
    e_s, negw_s, slot_s = lax.sort((ef, -wf, slot), num_keys=2)
    pos = jnp.arange(t * TOP_K, dtype=jnp.int32)
    is_start = jnp.concatenate([jnp.ones((1,), bool), e_s[1:] != e_s[:-1]])
    start_pos = lax.cummax(jnp.where(is_start, pos, 0))
    rank = pos - start_pos
    keep = rank < CAPACITY
    dest = jnp.where(keep, e_s * CAPACITY + rank, jnp.int32(2 ** 30))
    tok_all = jnp.zeros((num_experts * CAPACITY,), jnp.int32).at[dest].set(slot_s // TOP_K)
    val_all = jnp.zeros((num_experts * CAPACITY,), jnp.float32).at[dest].set(-negw_s)
    return tok_all, val_all


def kernel(hidden_states, ln1_w, qkv_w, q_norm_w, k_norm_w, o_proj_w, ln2_w,
           gate_w, w_gate_up, w_down, positions):
    t, d = hidden_states.shape
    n_qkv = qkv_w.shape[0]
    q_size = o_proj_w.shape[1]
    kv_size = (n_qkv - q_size) // 2
    head_dim = q_norm_w.shape[0]
    num_heads = q_size // head_dim
    num_kv = kv_size // head_dim
    rep = num_heads // num_kv
    num_experts = gate_w.shape[0]
    ff = w_gate_up.shape[2] // 2
    f32 = jnp.float32

    qkv = pl.pallas_call(
        _qkv_body,
        grid=(n_qkv // BN_QKV,),
        in_specs=[
            pl.BlockSpec((t, d), lambda j: (0, 0)),
            pl.BlockSpec((1, d), lambda j: (0, 0)),
            pl.BlockSpec((BN_QKV, d), lambda j: (j, 0)),
        ],
        out_specs=pl.BlockSpec((t, BN_QKV), lambda j: (0, j)),
        out_shape=jax.ShapeDtypeStruct((t, n_qkv), f32),
    )(hidden_states, ln1_w.reshape(1, d), qkv_w.astype(jnp.bfloat16))

    inv_freq = 1.0 / (ROPE_THETA ** (jnp.arange(0, head_dim, 2,
                                                dtype=jnp.float32) / head_dim))
    freqs = positions.astype(f32)[:, None] * inv_freq[None, :]
    cos_t = jnp.cos(freqs)
    sin_t = jnp.sin(freqs)

    k = qkv[:, q_size:q_size + kv_size].reshape(t, num_kv, head_dim)
    v2 = qkv[:, q_size + kv_size:].astype(jnp.bfloat16)
    k = _rope(positions, _rmsnorm(k, k_norm_w), head_dim).reshape(t, kv_size).astype(jnp.bfloat16)

    attn = pl.pallas_call(
        functools.partial(_attn_body, scale=head_dim ** -0.5, bq=BQ, t=t,
                          hd=head_dim),
        grid=(num_heads, t // BQ),
        in_specs=[
            pl.BlockSpec((BQ, head_dim), lambda h, i: (i, h)),
            pl.BlockSpec((1, head_dim), lambda h, i: (0, 0)),
            pl.BlockSpec((BQ, head_dim // 2), lambda h, i: (i, 0)),
            pl.BlockSpec((BQ, head_dim // 2), lambda h, i: (i, 0)),
            pl.BlockSpec((t, head_dim), lambda h, i: (0, h // rep)),
            pl.BlockSpec((t, head_dim), lambda h, i: (0, h // rep)),
        ],
        out_specs=pl.BlockSpec((BQ, head_dim), lambda h, i: (i, h)),
        out_shape=jax.ShapeDtypeStruct((t, q_size), jnp.bfloat16),
    )(qkv, q_norm_w.reshape(1, head_dim), cos_t, sin_t, k, v2)

    h_attn = pl.pallas_call(
        _oproj_body,
        grid=(t // BM_O,),
        in_specs=[
            pl.BlockSpec((BM_O, q_size), lambda i: (i, 0)),
            pl.BlockSpec((d, q_size), lambda i: (0, 0)),
            pl.BlockSpec((BM_O, d), lambda i: (i, 0)),
        ],
        out_specs=pl.BlockSpec((BM_O, d), lambda i: (i, 0)),
        out_shape=jax.ShapeDtypeStruct((t, d), f32),
    )(attn, o_proj_w.astype(jnp.bfloat16), hidden_states)

    h2, logits = pl.pallas_call(
        _normgate_body,
        grid=(1,),
        in_specs=[
            pl.BlockSpec((t, d), lambda j: (0, 0)),
            pl.BlockSpec((1, d), lambda j: (0, 0)),
            pl.BlockSpec((num_experts, d), lambda j: (0, 0)),
        ],
        out_specs=[
            pl.BlockSpec((t, d), lambda j: (0, 0)),
            pl.BlockSpec((t, num_experts), lambda j: (0, 0)),
        ],
        out_shape=[
            jax.ShapeDtypeStruct((t, d), f32),
            jax.ShapeDtypeStruct((t, num_experts), f32),
        ],
    )(h_attn, ln2_w.reshape(1, d), gate_w)

    tok_all, val_all = _route(logits, num_experts)
    tok4 = tok_all.reshape(num_experts, 1, 1, CAPACITY)
    val4 = val_all.reshape(num_experts, 1, 1, CAPACITY)

    h2b = h2.astype(jnp.bfloat16)
    ffc = ff // 2

    moe = pl.pallas_call(
        functools.partial(_moe_body, t=t),
        grid=(num_experts, 2),
        in_specs=[
            pl.BlockSpec((1, 1, 1, CAPACITY), lambda e, j: (e, 0, 0, 0)),
            pl.BlockSpec((1, 1, 1, CAPACITY), lambda e, j: (e, 0, 0, 0)),
            pl.BlockSpec((t, d), lambda e, j: (0, 0)),
            pl.BlockSpec((1, d, ffc), lambda e, j: (e, 0, j)),
            pl.BlockSpec((1, d, ffc), lambda e, j: (e, 0, 2 + j)),
            pl.BlockSpec((1, ffc, d), lambda e, j: (e, j, 0)),
        ],
        out_specs=pl.BlockSpec((t, d), lambda e, j: (0, 0)),
        out_shape=jax.ShapeDtypeStruct((t, d), f32),
        scratch_shapes=[
            pltpu.VMEM((CAPACITY, d), jnp.bfloat16),
            pltpu.VMEM((CAPACITY, d), jnp.float32),
        ],
    )(tok4, val4, h2b, w_gate_up, w_gate_up, w_down)

    return moe + h_attn
